# baseline (device time: 32682 ns/iter reference)
import jax
import jax.numpy as jnp
from jax import lax
from jax.experimental import pallas as pl
from jax.experimental.pallas import tpu as pltpu

N_DEV = 8
N_TOK = 512
D_IN = 256
D_OUT = 512
N_EXP = 32
E_LOCAL = N_EXP // N_DEV
ROWS = N_TOK // N_DEV


def kernel(x, router_W, route_idx, expert_W):
    def body(x_ref, rw_ref, ri_ref, ew_ref, out_ref,
             partial_ref, send_ref, recv_ref, send_sems, recv_sems):
        my = lax.axis_index("i")
        left = lax.rem(my + N_DEV - 1, N_DEV)
        right = lax.rem(my + 1, N_DEV)

        barrier_sem = pltpu.get_barrier_semaphore()
        for nbr in (left, right):
            pl.semaphore_signal(
                barrier_sem, inc=1,
                device_id=(nbr,), device_id_type=pl.DeviceIdType.MESH,
            )
        pl.semaphore_wait(barrier_sem, 2)

        xv = x_ref[:, :]
        scores = jnp.dot(xv, rw_ref[:, :], preferred_element_type=jnp.float32)
        m = jnp.max(scores, axis=-1, keepdims=True)
        p = jnp.exp(scores - m)
        p = p / jnp.sum(p, axis=-1, keepdims=True)

        ri = ri_ref[:, :]
        ri0 = ri[:, 0:1]
        ri1 = ri[:, 1:2]
        e_iota = lax.broadcasted_iota(jnp.int32, (N_TOK, N_EXP), 1)
        g0 = jnp.sum(jnp.where(e_iota == ri0, p, 0.0), axis=-1, keepdims=True)
        g1 = jnp.sum(jnp.where(e_iota == ri1, p, 0.0), axis=-1, keepdims=True)
        gs = g0 + g1

        acc = jnp.zeros((N_TOK, D_OUT), jnp.float32)
        for le in range(E_LOCAL):
            e = my * E_LOCAL + le
            w = (jnp.where(ri0 == e, g0, 0.0)
                 + jnp.where(ri1 == e, g1, 0.0)) / gs
            acc = acc + jnp.dot(
                xv * w, ew_ref[le, :, :], preferred_element_type=jnp.float32
            )
        partial_ref[:, :] = acc

        for s in range(N_DEV - 1):
            c = lax.rem(my + N_DEV - 1 - s, N_DEV)
            if s == 0:
                send_ref[s, :, :] = partial_ref[pl.ds(c * ROWS, ROWS), :]
            else:
                send_ref[s, :, :] = (
                    recv_ref[s - 1, :, :]
                    + partial_ref[pl.ds(c * ROWS, ROWS), :]
                )
            rdma = pltpu.make_async_remote_copy(
                src_ref=send_ref.at[s],
                dst_ref=recv_ref.at[s],
                send_sem=send_sems.at[s],
                recv_sem=recv_sems.at[s],
                device_id=(right,),
                device_id_type=pl.DeviceIdType.MESH,
            )
            rdma.start()
            rdma.wait()

        out_ref[:, :] = (
            recv_ref[N_DEV - 2, :, :] + partial_ref[pl.ds(my * ROWS, ROWS), :]
        )

    return pl.pallas_call(
        body,
        out_shape=jax.ShapeDtypeStruct((ROWS, D_OUT), jnp.float32),
        in_specs=[pl.BlockSpec(memory_space=pltpu.VMEM)] * 4,
        out_specs=pl.BlockSpec(memory_space=pltpu.VMEM),
        scratch_shapes=[
            pltpu.VMEM((N_TOK, D_OUT), jnp.float32),
            pltpu.VMEM((N_DEV - 1, ROWS, D_OUT), jnp.float32),
            pltpu.VMEM((N_DEV - 1, ROWS, D_OUT), jnp.float32),
            pltpu.SemaphoreType.DMA((N_DEV - 1,)),
            pltpu.SemaphoreType.DMA((N_DEV - 1,)),
        ],
        compiler_params=pltpu.CompilerParams(collective_id=0),
    )(x, router_W, route_idx, expert_W)


# device time: 18720 ns/iter; 1.7458x vs baseline; 1.7458x over previous
import jax
import jax.numpy as jnp
from jax import lax
from jax.experimental import pallas as pl
from jax.experimental.pallas import tpu as pltpu

N_DEV = 8
N_TOK = 512
D_IN = 256
D_OUT = 512
N_EXP = 32
E_LOCAL = N_EXP // N_DEV
ROWS = N_TOK // N_DEV


def kernel(x, router_W, route_idx, expert_W):
    def body(x_ref, rw_ref, ri_ref, ew_ref, out_ref,
             partial_ref, recv_ref, send_sems, recv_sems):
        my = lax.axis_index("i")

        barrier_sem = pltpu.get_barrier_semaphore()
        for t in range(1, N_DEV):
            pl.semaphore_signal(
                barrier_sem, inc=1,
                device_id=(lax.rem(my + t, N_DEV),),
                device_id_type=pl.DeviceIdType.MESH,
            )
        pl.semaphore_wait(barrier_sem, N_DEV - 1)

        xv = x_ref[:, :]
        scores = jnp.dot(xv, rw_ref[:, :], preferred_element_type=jnp.float32)
        mx = jnp.max(scores, axis=-1, keepdims=True)
        p = jnp.exp(scores - mx)
        p = p / jnp.sum(p, axis=-1, keepdims=True)
        ri = ri_ref[:, :]
        ri0 = ri[:, 0:1]
        ri1 = ri[:, 1:2]
        e_iota = lax.broadcasted_iota(jnp.int32, (N_TOK, N_EXP), 1)
        g0 = jnp.sum(jnp.where(e_iota == ri0, p, 0.0), axis=-1, keepdims=True)
        g1 = jnp.sum(jnp.where(e_iota == ri1, p, 0.0), axis=-1, keepdims=True)
        gs = g0 + g1

        acc = jnp.zeros((N_TOK, D_OUT), jnp.float32)
        for le in range(E_LOCAL):
            e = my * E_LOCAL + le
            w = (jnp.where(ri0 == e, g0, 0.0)
                 + jnp.where(ri1 == e, g1, 0.0)) / gs
            acc = acc + jnp.dot(
                xv * w, ew_ref[le, :, :], preferred_element_type=jnp.float32
            )
        partial_ref[:, :] = acc

        sends = []
        for t in range(1, N_DEV):
            k = lax.rem(my + t, N_DEV)
            rdma = pltpu.make_async_remote_copy(
                src_ref=partial_ref.at[pl.ds(k * ROWS, ROWS), :],
                dst_ref=recv_ref.at[N_DEV - 1 - t],
                send_sem=send_sems.at[t - 1],
                recv_sem=recv_sems.at[N_DEV - 1 - t],
                device_id=(k,),
                device_id_type=pl.DeviceIdType.MESH,
            )
            rdma.start()
            sends.append(rdma)

        out = partial_ref[pl.ds(my * ROWS, ROWS), :]
        for t in range(1, N_DEV):
            s = N_DEV - 1 - t
            recv = pltpu.make_async_remote_copy(
                src_ref=partial_ref.at[pl.ds(0, ROWS), :],
                dst_ref=recv_ref.at[s],
                send_sem=send_sems.at[0],
                recv_sem=recv_sems.at[s],
                device_id=(my,),
                device_id_type=pl.DeviceIdType.MESH,
            )
            recv.wait_recv()
            out = out + recv_ref[s, :, :]
        out_ref[:, :] = out

        for rdma in sends:
            rdma.wait_send()

    return pl.pallas_call(
        body,
        out_shape=jax.ShapeDtypeStruct((ROWS, D_OUT), jnp.float32),
        in_specs=[pl.BlockSpec(memory_space=pltpu.VMEM)] * 4,
        out_specs=pl.BlockSpec(memory_space=pltpu.VMEM),
        scratch_shapes=[
            pltpu.VMEM((N_TOK, D_OUT), jnp.float32),
            pltpu.VMEM((N_DEV - 1, ROWS, D_OUT), jnp.float32),
            pltpu.SemaphoreType.DMA((N_DEV - 1,)),
            pltpu.SemaphoreType.DMA((N_DEV - 1,)),
        ],
        compiler_params=pltpu.CompilerParams(collective_id=0),
    )(x, router_W, route_idx, expert_W)
